# baseline (speedup 1.0000x reference)
import functools
import jax
import jax.numpy as jnp
from jax import lax
from jax.experimental import pallas as pl
from jax.experimental.pallas import tpu as pltpu
from jax.experimental.pallas import tpu_sc as plsc

H = 2048
E = 16
I = 1024
ALPHA = 1.702
LIMIT = 7.0
EPS = 1e-5

T = 32
C = 2
HW = H // C


def _prologue_tc(x_ref, scale_ref, wg_ref, bg_ref, logits_ref):
    xx = x_ref[...]
    h = xx * jax.lax.rsqrt(jnp.mean(xx * xx, axis=-1, keepdims=True) + EPS)
    h = h * scale_ref[...]
    logits_ref[...] = jnp.dot(h, wg_ref[...],
                              preferred_element_type=jnp.float32) + bg_ref[...]


def _gather16(v, idx):
    dnums = lax.GatherDimensionNumbers(
        offset_dims=(), collapsed_slice_dims=(0,), start_index_map=(0,))
    return lax.gather(v, idx[:, None], dnums, slice_sizes=(1,),
                      mode=lax.GatherScatterMode.PROMISE_IN_BOUNDS)


def _bfly(v, op):
    iota = lax.iota(jnp.int32, 16)
    for s in (8, 4, 2, 1):
        v = op(v, _gather16(v, iota ^ s))
    return v


def _gates_sc(logits_hbm, gates_hbm, lvec, gvec):
    wid = lax.axis_index("s") * 2 + lax.axis_index("c")
    pltpu.sync_copy(logits_hbm.at[wid], lvec)
    lo = lvec[...]
    iota = lax.iota(jnp.int32, 16)
    big = jnp.broadcast_to(jnp.int32(E), (16,))
    m1 = _bfly(lo, jnp.maximum)
    i1 = _bfly(jnp.where(lo == m1, iota, big), jnp.minimum)
    masked = jnp.where(iota == i1, -1e30, lo)
    m2 = _bfly(masked, jnp.maximum)
    i2 = _bfly(jnp.where(masked == m2, iota, big), jnp.minimum)
    p1 = 1.0 / (1.0 + jnp.exp(m2 - m1))
    zero = jnp.zeros((16,), jnp.float32)
    gvec[...] = jnp.where(iota == i1, p1, zero) + jnp.where(iota == i2, 1.0 - p1, zero)
    pltpu.sync_copy(gvec, gates_hbm.at[wid])


def _moe_step(x_ref, scale_ref, gates_in_ref, w1_ref, b1_ref, w2_ref, b2_ref,
              out_ref, h_ref, gates_ref, a_ref, sg_ref):
    e = pl.program_id(0)
    c = pl.program_id(1)

    @pl.when((e == 0) & (c == 0))
    def _init():
        xx = x_ref[...]
        h = xx * jax.lax.rsqrt(jnp.mean(xx * xx, axis=-1, keepdims=True) + EPS)
        h = h * scale_ref[...]
        for k in range(C):
            h_ref[k] = h[:, k * HW:(k + 1) * HW].astype(jnp.bfloat16)
        gates_ref[...] = gates_in_ref[...]
        out_ref[...] = xx
        r = jax.lax.broadcasted_iota(jnp.int32, (2 * I, I), 0)
        j = jax.lax.broadcasted_iota(jnp.int32, (2 * I, I), 1)
        sg_ref[...] = (r == 2 * j).astype(jnp.bfloat16)

    partial = jnp.dot(h_ref[c], w1_ref[0].astype(jnp.bfloat16),
                      preferred_element_type=jnp.float32)

    @pl.when(c == 0)
    def _store_a():
        a_ref[...] = partial

    @pl.when(c == C - 1)
    def _finish_expert():
        a = a_ref[...] + partial
        af = a + b1_ref[0]
        g = jnp.minimum(af, LIMIT)
        gact = g * (1.0 / (1.0 + jnp.exp(-ALPHA * g)))
        lact = jnp.clip(af, -LIMIT, LIMIT) + 1.0
        lane = jax.lax.broadcasted_iota(jnp.int32, (T, 2 * I), 1)
        act = jnp.where(lane % 2 == 0, gact, lact)
        v = act * pltpu.roll(act, 2 * I - 1, axis=1)
        u = jnp.dot(v.astype(jnp.bfloat16), sg_ref[...],
                    preferred_element_type=jnp.float32)
        iota_e = jax.lax.broadcasted_iota(jnp.int32, (T, E), 1)
        gcol = jnp.sum(jnp.where(iota_e == e, gates_ref[...], 0.0),
                       axis=-1, keepdims=True)
        down = jnp.dot((u * gcol).astype(jnp.bfloat16), w2_ref[0].astype(jnp.bfloat16),
                       preferred_element_type=jnp.float32)
        out_ref[...] += down + gcol * b2_ref[0]


def kernel(x, scale, wg, bg, w1, b1, w2, b2):
    shape = x.shape
    x2 = x.reshape(T, H)

    logits = pl.pallas_call(
        _prologue_tc,
        out_shape=jax.ShapeDtypeStruct((T, E), jnp.float32),
    )(x2, scale.reshape(1, H), wg, bg.reshape(1, E))

    mesh = plsc.VectorSubcoreMesh(core_axis_name="c", subcore_axis_name="s")
    gates = functools.partial(
        pl.kernel, mesh=mesh,
        out_type=jax.ShapeDtypeStruct((T, E), jnp.float32),
        scratch_types=[
            pltpu.VMEM((E,), jnp.float32),
            pltpu.VMEM((E,), jnp.float32),
        ],
    )(_gates_sc)(logits)

    y = pl.pallas_call(
        _moe_step,
        grid=(E, C),
        in_specs=[
            pl.BlockSpec((T, H), lambda e, c: (0, 0)),
            pl.BlockSpec((1, H), lambda e, c: (0, 0)),
            pl.BlockSpec((T, E), lambda e, c: (0, 0)),
            pl.BlockSpec((1, HW, 2 * I), lambda e, c: (e, c, 0)),
            pl.BlockSpec((1, 1, 2 * I), lambda e, c: (e, 0, 0)),
            pl.BlockSpec((1, I, H), lambda e, c: (e, 0, 0)),
            pl.BlockSpec((1, 1, H), lambda e, c: (e, 0, 0)),
        ],
        out_specs=pl.BlockSpec((T, H), lambda e, c: (0, 0)),
        out_shape=jax.ShapeDtypeStruct((T, H), jnp.float32),
        scratch_shapes=[
            pltpu.VMEM((C, T, HW), jnp.bfloat16),
            pltpu.VMEM((T, E), jnp.float32),
            pltpu.VMEM((T, 2 * I), jnp.float32),
            pltpu.VMEM((2 * I, I), jnp.bfloat16),
        ],
    )(x2, scale.reshape(1, H), gates,
      w1, b1.reshape(E, 1, 2 * I), w2, b2.reshape(E, 1, H))
    return y.reshape(shape)


# speedup vs baseline: 1.1238x; 1.1238x over previous
import jax
import jax.numpy as jnp
from jax.experimental import pallas as pl
from jax.experimental.pallas import tpu as pltpu

H = 2048
E = 16
I = 1024
ALPHA = 1.702
LIMIT = 7.0
EPS = 1e-5

T = 32
C = 2
HW = H // C


def _moe_step(x_ref, scale_ref, wg_ref, bg_ref, w1_ref, b1_ref, w2_ref, b2_ref,
              out_ref, h_ref, gates_ref, a_ref, sg_ref):
    e = pl.program_id(0)
    c = pl.program_id(1)

    @pl.when((e == 0) & (c == 0))
    def _routing():
        xx = x_ref[...]
        h = xx * jax.lax.rsqrt(jnp.mean(xx * xx, axis=-1, keepdims=True) + EPS)
        h = h * scale_ref[...]
        for k in range(C):
            h_ref[k] = h[:, k * HW:(k + 1) * HW].astype(jnp.bfloat16)
        logits = jnp.dot(h, wg_ref[...], preferred_element_type=jnp.float32)
        logits = logits + bg_ref[...]
        iota = jax.lax.broadcasted_iota(jnp.int32, (T, E), 1)
        m1 = jnp.max(logits, axis=-1, keepdims=True)
        i1 = jnp.min(jnp.where(logits == m1, iota, E), axis=-1, keepdims=True)
        masked = jnp.where(iota == i1, -jnp.inf, logits)
        m2 = jnp.max(masked, axis=-1, keepdims=True)
        i2 = jnp.min(jnp.where(masked == m2, iota, E), axis=-1, keepdims=True)
        p1 = 1.0 / (1.0 + jnp.exp(m2 - m1))
        gates_ref[...] = jnp.where(iota == i1, p1, 0.0) + jnp.where(iota == i2, 1.0 - p1, 0.0)
        out_ref[...] = xx
        r = jax.lax.broadcasted_iota(jnp.int32, (2 * I, I), 0)
        j = jax.lax.broadcasted_iota(jnp.int32, (2 * I, I), 1)
        sg_ref[...] = (r == 2 * j).astype(jnp.bfloat16)

    partial = jnp.dot(h_ref[c], w1_ref[0].astype(jnp.bfloat16),
                      preferred_element_type=jnp.float32)

    @pl.when(c == 0)
    def _store_a():
        a_ref[...] = partial

    @pl.when(c == C - 1)
    def _finish_expert():
        a = a_ref[...] + partial
        af = a + b1_ref[0]
        g = jnp.minimum(af, LIMIT)
        gact = g * (1.0 / (1.0 + jnp.exp(-ALPHA * g)))
        lact = jnp.clip(af, -LIMIT, LIMIT) + 1.0
        lane = jax.lax.broadcasted_iota(jnp.int32, (T, 2 * I), 1)
        act = jnp.where(lane % 2 == 0, gact, lact)
        v = act * pltpu.roll(act, 2 * I - 1, axis=1)
        u = jnp.dot(v.astype(jnp.bfloat16), sg_ref[...],
                    preferred_element_type=jnp.float32)
        iota_e = jax.lax.broadcasted_iota(jnp.int32, (T, E), 1)
        gcol = jnp.sum(jnp.where(iota_e == e, gates_ref[...], 0.0),
                       axis=-1, keepdims=True)
        down = jnp.dot((u * gcol).astype(jnp.bfloat16), w2_ref[0].astype(jnp.bfloat16),
                       preferred_element_type=jnp.float32)
        out_ref[...] += down + gcol * b2_ref[0]


def kernel(x, scale, wg, bg, w1, b1, w2, b2):
    shape = x.shape
    x2 = x.reshape(T, H)
    y = pl.pallas_call(
        _moe_step,
        grid=(E, C),
        in_specs=[
            pl.BlockSpec((T, H), lambda e, c: (0, 0)),
            pl.BlockSpec((1, H), lambda e, c: (0, 0)),
            pl.BlockSpec((H, E), lambda e, c: (0, 0)),
            pl.BlockSpec((1, E), lambda e, c: (0, 0)),
            pl.BlockSpec((1, HW, 2 * I), lambda e, c: (e, c, 0)),
            pl.BlockSpec((1, 1, 2 * I), lambda e, c: (e, 0, 0)),
            pl.BlockSpec((1, I, H), lambda e, c: (e, 0, 0)),
            pl.BlockSpec((1, 1, H), lambda e, c: (e, 0, 0)),
        ],
        out_specs=pl.BlockSpec((T, H), lambda e, c: (0, 0)),
        out_shape=jax.ShapeDtypeStruct((T, H), jnp.float32),
        scratch_shapes=[
            pltpu.VMEM((C, T, HW), jnp.bfloat16),
            pltpu.VMEM((T, E), jnp.float32),
            pltpu.VMEM((T, 2 * I), jnp.float32),
            pltpu.VMEM((2 * I, I), jnp.bfloat16),
        ],
    )(x2, scale.reshape(1, H), wg, bg.reshape(1, E),
      w1, b1.reshape(E, 1, 2 * I), w2, b2.reshape(E, 1, H))
    return y.reshape(shape)


# speedup vs baseline: 1.1302x; 1.0057x over previous
import jax
import jax.numpy as jnp
from jax.experimental import pallas as pl
from jax.experimental.pallas import tpu as pltpu

H = 2048
E = 16
I = 1024
ALPHA = 1.702
LIMIT = 7.0
EPS = 1e-5

T = 32
C = 2
CW = 2 * I // C
IW = CW // 2


def _moe_step(x_ref, scale_ref, wg_ref, bg_ref, w1_ref, b1_ref, w2_ref, b2_ref,
              out_ref, h_ref, gates_ref, sg_ref):
    e = pl.program_id(0)
    c = pl.program_id(1)

    @pl.when((e == 0) & (c == 0))
    def _routing():
        xx = x_ref[...]
        h = xx * jax.lax.rsqrt(jnp.mean(xx * xx, axis=-1, keepdims=True) + EPS)
        h = h * scale_ref[...]
        h_ref[...] = h.astype(jnp.bfloat16)
        logits = jnp.dot(h, wg_ref[...], preferred_element_type=jnp.float32)
        logits = logits + bg_ref[...]
        iota = jax.lax.broadcasted_iota(jnp.int32, (T, E), 1)
        m1 = jnp.max(logits, axis=-1, keepdims=True)
        i1 = jnp.min(jnp.where(logits == m1, iota, E), axis=-1, keepdims=True)
        masked = jnp.where(iota == i1, -jnp.inf, logits)
        m2 = jnp.max(masked, axis=-1, keepdims=True)
        i2 = jnp.min(jnp.where(masked == m2, iota, E), axis=-1, keepdims=True)
        p1 = 1.0 / (1.0 + jnp.exp(m2 - m1))
        gates_ref[...] = jnp.where(iota == i1, p1, 0.0) + jnp.where(iota == i2, 1.0 - p1, 0.0)
        out_ref[...] = xx
        r = jax.lax.broadcasted_iota(jnp.int32, (CW, IW), 0)
        j = jax.lax.broadcasted_iota(jnp.int32, (CW, IW), 1)
        sg_ref[...] = (r == 2 * j).astype(jnp.bfloat16)

    a = jnp.dot(h_ref[...], w1_ref[0].astype(jnp.bfloat16),
                preferred_element_type=jnp.float32)
    af = a + b1_ref[0]
    g = jnp.minimum(af, LIMIT)
    gact = g * (1.0 / (1.0 + jnp.exp(-ALPHA * g)))
    lact = jnp.clip(af, -LIMIT, LIMIT) + 1.0
    lane = jax.lax.broadcasted_iota(jnp.int32, (T, CW), 1)
    act = jnp.where(lane % 2 == 0, gact, lact)
    v = act * pltpu.roll(act, CW - 1, axis=1)
    u = jnp.dot(v.astype(jnp.bfloat16), sg_ref[...],
                preferred_element_type=jnp.float32)
    iota_e = jax.lax.broadcasted_iota(jnp.int32, (T, E), 1)
    gcol = jnp.sum(jnp.where(iota_e == e, gates_ref[...], 0.0),
                   axis=-1, keepdims=True)
    down = jnp.dot((u * gcol).astype(jnp.bfloat16), w2_ref[0].astype(jnp.bfloat16),
                   preferred_element_type=jnp.float32)
    out_ref[...] += down + jnp.where(c == 0, 1.0, 0.0) * (gcol * b2_ref[0])


def kernel(x, scale, wg, bg, w1, b1, w2, b2):
    shape = x.shape
    x2 = x.reshape(T, H)
    y = pl.pallas_call(
        _moe_step,
        grid=(E, C),
        in_specs=[
            pl.BlockSpec((T, H), lambda e, c: (0, 0)),
            pl.BlockSpec((1, H), lambda e, c: (0, 0)),
            pl.BlockSpec((H, E), lambda e, c: (0, 0)),
            pl.BlockSpec((1, E), lambda e, c: (0, 0)),
            pl.BlockSpec((1, H, CW), lambda e, c: (e, 0, c)),
            pl.BlockSpec((1, 1, CW), lambda e, c: (e, 0, c)),
            pl.BlockSpec((1, IW, H), lambda e, c: (e, c, 0)),
            pl.BlockSpec((1, 1, H), lambda e, c: (e, 0, 0)),
        ],
        out_specs=pl.BlockSpec((T, H), lambda e, c: (0, 0)),
        out_shape=jax.ShapeDtypeStruct((T, H), jnp.float32),
        scratch_shapes=[
            pltpu.VMEM((T, H), jnp.bfloat16),
            pltpu.VMEM((T, E), jnp.float32),
            pltpu.VMEM((CW, IW), jnp.bfloat16),
        ],
    )(x2, scale.reshape(1, H), wg, bg.reshape(1, E),
      w1, b1.reshape(E, 1, 2 * I), w2, b2.reshape(E, 1, H))
    return y.reshape(shape)
